# baseline (speedup 1.0000x reference)
import jax
import jax.numpy as jnp
from jax import lax
from jax.experimental import pallas as pl
from jax.experimental.pallas import tpu as pltpu
from jax.experimental.pallas import tpu_sc as plsc

N = 10000
NP = 10240
D = 128
NC = 2
NS = 16
NW = NC * NS
CHUNK = 128
T = 80
E_PAD = NW * T * CHUNK
NGRP = 4
GRP = NP // NGRP
G_ACC = GRP + CHUNK
G_ROWS = G_ACC // NS
NBUF = 4
RB = 640
TCAP = T + NBUF
STG = T * CHUNK + 1280

_SC_PARAMS = pltpu.CompilerParams(use_tc_tiling_on_sc=False,
                                  needs_layout_passes=False)


def _mesh():
    return plsc.VectorSubcoreMesh(core_axis_name="c", subcore_axis_name="s")


def _prep_body(src_hbm, dst_hbm, deg_out, srcL, dstL, cnt_out,
               src_v, dst_v, stage_s, stage_d, cnt_v, ones_v, zero_v, acc_sh):
    c = lax.axis_index("c")
    s = lax.axis_index("s")
    w = c * NS + s
    lane = lax.iota(jnp.int32, 16)

    tbase = c * (NS * T) + s * T
    pltpu.sync_copy(src_hbm.at[pl.ds(tbase, T)], src_v)
    pltpu.sync_copy(dst_hbm.at[pl.ds(tbase, T)], dst_v)

    row0 = s * (NP // NS)

    def zfill(i, carry):
        zero_v[pl.ds(i * 16, 16)] = jnp.zeros((16,), jnp.float32)
        return carry

    lax.fori_loop(0, (NP // NS) // 16, zfill, 0)
    pltpu.sync_copy(zero_v, acc_sh.at[pl.ds(row0, NP // NS)])

    def ofill(i, carry):
        ones_v[pl.ds(i * 16, 16)] = jnp.ones((16,), jnp.float32)
        return carry

    lax.fori_loop(0, CHUNK // 16, ofill, 0)

    plsc.subcore_barrier()

    def dbody(j, carry):
        pltpu.sync_copy(ones_v, acc_sh.at[dst_v.at[j]], add=True)
        return carry

    lax.fori_loop(0, T, dbody, 0)

    plsc.subcore_barrier()
    pltpu.sync_copy(acc_sh.at[pl.ds(row0, NP // NS)],
                    deg_out.at[pl.ds(c * NP + row0, NP // NS)])

    cnt_v[...] = jnp.zeros((16,), jnp.int32)

    for g in range(NGRP):
        gbase = g * GRP

        def cbody(i, off_vec):
            r = i // (CHUNK // 16)
            k = i % (CHUNK // 16)
            vd = dst_v[r, pl.ds(k * 16, 16)] - gbase
            m = (vd >= 0) & (vd < GRP)
            vs = src_v[r, pl.ds(k * 16, 16)]
            cum = plsc.cumsum(m.astype(jnp.int32))
            pos = jnp.where(m, off_vec + cum - 1, STG - 16 + lane)
            plsc.store_scatter(stage_d, [pos], vd)
            plsc.store_scatter(stage_s, [pos], vs)
            return off_vec + plsc.all_reduce_population_count(m)

        off_vec = lax.fori_loop(0, T * (CHUNK // 16), cbody,
                                jnp.zeros((16,), jnp.int32))
        off = lax.reduce_max(off_vec, (0,))

        v0 = off // 16
        rem = off - v0 * 16
        dmp = GRP + lane
        keep = lane < rem
        pv_d = stage_d[pl.ds(v0 * 16, 16)]
        pv_s = stage_s[pl.ds(v0 * 16, 16)]
        stage_d[pl.ds(v0 * 16, 16)] = jnp.where(keep, pv_d, dmp)
        stage_s[pl.ds(v0 * 16, 16)] = jnp.where(keep, pv_s, 0)

        def pfill(i, carry):
            base = (v0 + 1 + i) * 16
            stage_d[pl.ds(base, 16)] = dmp
            stage_s[pl.ds(base, 16)] = jnp.zeros((16,), jnp.int32)
            return carry

        lax.fori_loop(0, 66, pfill, 0)

        nch = (off + CHUNK - 1) // CHUNK
        ng = jnp.maximum((nch + NBUF - 1) // NBUF, 1)
        rowbase = (w * NGRP + g) * TCAP

        def cpy(j, carry):
            pltpu.sync_copy(stage_s.at[pl.ds(j * CHUNK, CHUNK)],
                            srcL.at[rowbase + j])
            pltpu.sync_copy(stage_d.at[pl.ds(j * CHUNK, CHUNK)],
                            dstL.at[rowbase + j])
            return carry

        lax.fori_loop(0, ng * NBUF + NBUF, cpy, 0)

        cnt_v[...] = jnp.where(lane == g, ng, cnt_v[...])

    pltpu.sync_copy(cnt_v, cnt_out.at[w])


def _prep_call(src_p, dst_p):
    fn = pl.kernel(
        _prep_body,
        out_type=[
            jax.ShapeDtypeStruct((NC * NP,), jnp.float32),
            jax.ShapeDtypeStruct((NW * NGRP * TCAP, CHUNK), jnp.int32),
            jax.ShapeDtypeStruct((NW * NGRP * TCAP, CHUNK), jnp.int32),
            jax.ShapeDtypeStruct((NW, 16), jnp.int32),
        ],
        mesh=_mesh(),
        compiler_params=_SC_PARAMS,
        scratch_types=[
            pltpu.VMEM((T, CHUNK), jnp.int32),
            pltpu.VMEM((T, CHUNK), jnp.int32),
            pltpu.VMEM((STG,), jnp.int32),
            pltpu.VMEM((STG,), jnp.int32),
            pltpu.VMEM((16,), jnp.int32),
            pltpu.VMEM((CHUNK,), jnp.float32),
            pltpu.VMEM((NP // NS,), jnp.float32),
            pltpu.VMEM_SHARED((NP,), jnp.float32),
        ],
    )
    return fn(src_p, dst_p)


def _agg_body(h_hbm, srcL, dstL, cnt_hbm, out_hbm, src_v, dst_v, cnt_v, rows,
              acc_sh, sem0, sem1, sem2, sem3):
    sems = [sem0, sem1, sem2, sem3]
    c = lax.axis_index("c")
    s = lax.axis_index("s")
    w = c * NS + s
    row0 = s * G_ROWS
    lane = lax.iota(jnp.int32, 16)

    pltpu.sync_copy(cnt_hbm.at[w], cnt_v)

    for g in range(NGRP):
        rowbase = (w * NGRP + g) * TCAP
        pltpu.sync_copy(srcL.at[pl.ds(rowbase, TCAP)], src_v)
        pltpu.sync_copy(dstL.at[pl.ds(rowbase, TCAP)], dst_v)

        def zfill(i, carry):
            r = i // (D // 16)
            k = i % (D // 16)
            rows[0, r, pl.ds(k * 16, 16)] = jnp.zeros((16,), jnp.float32)
            return carry

        lax.fori_loop(0, CHUNK * (D // 16), zfill, 0)
        for off in range(0, G_ROWS, CHUNK):
            ln = min(CHUNK, G_ROWS - off)
            pltpu.sync_copy(rows.at[0, pl.ds(0, ln)],
                            acc_sh.at[pl.ds(row0 + off, ln)])

        plsc.subcore_barrier()

        ng = lax.reduce_max(jnp.where(lane == g, cnt_v[...], 0), (0,))
        nch = ng * NBUF

        for b in range(NBUF):
            pltpu.async_copy(h_hbm.at[src_v.at[b]], rows.at[b], sems[b])

        def ring(gg, carry):
            for b in range(NBUF):
                j = gg * NBUF + b
                pltpu.make_async_copy(h_hbm.at[src_v.at[j]], rows.at[b],
                                      sems[b]).wait()
                pltpu.sync_copy(rows.at[b], acc_sh.at[dst_v.at[j]], add=True)
                pltpu.async_copy(h_hbm.at[src_v.at[j + NBUF]], rows.at[b],
                                 sems[b])

            return carry

        lax.fori_loop(0, ng, ring, 0)

        for b in range(NBUF):
            pltpu.make_async_copy(h_hbm.at[src_v.at[nch + b]], rows.at[b],
                                  sems[b]).wait()

        plsc.subcore_barrier()
        for off in range(0, G_ROWS, CHUNK):
            ln = min(CHUNK, G_ROWS - off)
            r = row0 + off
            pltpu.sync_copy(acc_sh.at[pl.ds(r, ln)],
                            out_hbm.at[c, g, pl.ds(r, ln)])


def _agg_call(h, srcL, dstL, cnt):
    fn = pl.kernel(
        _agg_body,
        out_type=jax.ShapeDtypeStruct((NC, NGRP, G_ACC, D), jnp.float32),
        mesh=_mesh(),
        compiler_params=_SC_PARAMS,
        scratch_types=[
            pltpu.VMEM((TCAP, CHUNK), jnp.int32),
            pltpu.VMEM((TCAP, CHUNK), jnp.int32),
            pltpu.VMEM((16,), jnp.int32),
            pltpu.VMEM((NBUF, CHUNK, D), jnp.float32),
            pltpu.VMEM_SHARED((G_ACC, D), jnp.float32),
            pltpu.SemaphoreType.DMA,
            pltpu.SemaphoreType.DMA,
            pltpu.SemaphoreType.DMA,
            pltpu.SemaphoreType.DMA,
        ],
    )
    return fn(h, srcL, dstL, cnt)


def _mm_body(cur_ref, w_ref, d_ref, hp_ref):
    xw = jnp.dot(cur_ref[...], w_ref[...], preferred_element_type=jnp.float32)
    hp_ref[...] = xw * d_ref[...]


def _mm_call(cur, w, d_bcast):
    return pl.pallas_call(
        _mm_body,
        grid=(NP // RB,),
        in_specs=[
            pl.BlockSpec((RB, D), lambda i: (i, 0)),
            pl.BlockSpec((D, D), lambda i: (0, 0)),
            pl.BlockSpec((RB, D), lambda i: (i, 0)),
        ],
        out_specs=pl.BlockSpec((RB, D), lambda i: (i, 0)),
        out_shape=jax.ShapeDtypeStruct((NP, D), jnp.float32),
    )(cur, w, d_bcast)


def _comb_body(agg_ref, hp_ref, d_ref, b_ref, f_ref, out_ref):
    t = (d_ref[...] * (agg_ref[0, 0] + agg_ref[1, 0] + hp_ref[...])
         + b_ref[...])
    out_ref[...] = jnp.where(f_ref[...] > 0.0, jnp.maximum(t, 0.0), t)


def _comb_call(agg, hp, d_bcast, b, flag):
    bpg = GRP // RB
    return pl.pallas_call(
        _comb_body,
        grid=(NP // RB,),
        in_specs=[
            pl.BlockSpec((NC, 1, RB, D),
                         lambda i: (0, i // bpg, i % bpg, 0)),
            pl.BlockSpec((RB, D), lambda i: (i, 0)),
            pl.BlockSpec((RB, D), lambda i: (i, 0)),
            pl.BlockSpec((1, D), lambda i: (0, 0)),
            pl.BlockSpec((1, D), lambda i: (0, 0)),
        ],
        out_specs=pl.BlockSpec((RB, D), lambda i: (i, 0)),
        out_shape=jax.ShapeDtypeStruct((NP, D), jnp.float32),
    )(agg, hp, d_bcast, b, flag)


def kernel(x, edge_index, W1, b1, W2, b2):
    src = edge_index[0]
    dst = edge_index[1]
    e = src.shape[0]
    pad = E_PAD - e
    src_p = jnp.concatenate(
        [src, jnp.zeros((pad,), src.dtype)]).reshape(E_PAD // CHUNK, CHUNK)
    dst_p = jnp.concatenate(
        [dst, jnp.full((pad,), N, dst.dtype)]).reshape(E_PAD // CHUNK, CHUNK)

    deg_parts, srcL, dstL, cnt = _prep_call(src_p, dst_p)
    deg = deg_parts[:NP] + deg_parts[NP:] + 1.0
    d_bcast = jnp.broadcast_to(lax.rsqrt(deg)[:, None], (NP, D))

    x_pad = jnp.concatenate(
        [x, jnp.zeros((NP - N, D), jnp.float32)], axis=0)

    def layer(it, cur):
        w = jnp.where(it == 0, W1, W2)
        b = jnp.where(it == 0, b1, b2).reshape(1, D)
        flag = jnp.where(it == 0, 1.0, 0.0) * jnp.ones((1, D), jnp.float32)
        hp = _mm_call(cur, w, d_bcast)
        agg = _agg_call(hp, srcL, dstL, cnt)
        return _comb_call(agg, hp, d_bcast, b, flag)

    return lax.fori_loop(0, 2, layer, x_pad)[:N]


# speedup vs baseline: 2.0047x; 2.0047x over previous
import jax
import jax.numpy as jnp
from jax import lax
from jax.experimental import pallas as pl
from jax.experimental.pallas import tpu as pltpu
from jax.experimental.pallas import tpu_sc as plsc

N = 10000
NP = 10240
D = 128
NC = 2
NS = 16
NW = NC * NS
CHUNK = 128
T = 80
E_PAD = NW * T * CHUNK
NGRP = 4
GRP = NP // NGRP
G_ACC = GRP + CHUNK
G_ROWS = G_ACC // NS
NBUF = 4
RB = 640
STG = T * CHUNK + 768

_SC_PARAMS = pltpu.CompilerParams(use_tc_tiling_on_sc=False,
                                  needs_layout_passes=False)


def _mesh():
    return plsc.VectorSubcoreMesh(core_axis_name="c", subcore_axis_name="s")


def _prep_body(src_hbm, dst_hbm, deg_out, srcL, dstL, cnt_out,
               src_v, dst_v, stage_s, stage_d, cnt_v, ones_v, zero_v, acc_sh):
    c = lax.axis_index("c")
    s = lax.axis_index("s")
    w = c * NS + s
    lane = lax.iota(jnp.int32, 16)

    tbase = c * (NS * T) + s * T
    pltpu.sync_copy(src_hbm.at[pl.ds(tbase, T)], src_v)
    pltpu.sync_copy(dst_hbm.at[pl.ds(tbase, T)], dst_v)

    row0 = s * (NP // NS)

    def zfill(i, carry):
        zero_v[pl.ds(i * 16, 16)] = jnp.zeros((16,), jnp.float32)
        return carry

    lax.fori_loop(0, (NP // NS) // 16, zfill, 0)
    pltpu.sync_copy(zero_v, acc_sh.at[pl.ds(row0, NP // NS)])

    def ofill(i, carry):
        ones_v[pl.ds(i * 16, 16)] = jnp.ones((16,), jnp.float32)
        return carry

    lax.fori_loop(0, CHUNK // 16, ofill, 0)

    plsc.subcore_barrier()

    def dbody(j, carry):
        pltpu.sync_copy(ones_v, acc_sh.at[dst_v.at[j]], add=True)
        return carry

    lax.fori_loop(0, T, dbody, 0)

    plsc.subcore_barrier()
    pltpu.sync_copy(acc_sh.at[pl.ds(row0, NP // NS)],
                    deg_out.at[pl.ds(c * NP + row0, NP // NS)])

    cnt_v[...] = jnp.zeros((16,), jnp.int32)

    for g in range(NGRP):
        gbase = g * GRP

        def cbody(i, off_vec):
            r = i // (CHUNK // 16)
            k = i % (CHUNK // 16)
            vd = dst_v[r, pl.ds(k * 16, 16)] - gbase
            m = (vd >= 0) & (vd < GRP)
            vs = src_v[r, pl.ds(k * 16, 16)]
            cum = plsc.cumsum(m.astype(jnp.int32))
            pos = jnp.where(m, off_vec + cum - 1, STG - 16 + lane)
            plsc.store_scatter(stage_d, [pos], vd)
            plsc.store_scatter(stage_s, [pos], vs)
            return off_vec + plsc.all_reduce_population_count(m)

        off_vec = lax.fori_loop(0, T * (CHUNK // 16), cbody,
                                jnp.zeros((16,), jnp.int32))
        off = lax.reduce_max(off_vec, (0,))

        v0 = off // 16
        rem = off - v0 * 16
        dmp = GRP + lane
        keep = lane < rem
        pv_d = stage_d[pl.ds(v0 * 16, 16)]
        pv_s = stage_s[pl.ds(v0 * 16, 16)]
        stage_d[pl.ds(v0 * 16, 16)] = jnp.where(keep, pv_d, dmp)
        stage_s[pl.ds(v0 * 16, 16)] = jnp.where(keep, pv_s, 0)

        def pfill(i, carry):
            base = (v0 + 1 + i) * 16
            stage_d[pl.ds(base, 16)] = dmp
            stage_s[pl.ds(base, 16)] = jnp.zeros((16,), jnp.int32)
            return carry

        lax.fori_loop(0, 34, pfill, 0)

        nch = (off + CHUNK - 1) // CHUNK
        ng = (nch + NBUF - 1) // NBUF
        rowbase = (w * NGRP + g) * T

        def cpy(j, carry):
            pltpu.sync_copy(stage_s.at[pl.ds(j * CHUNK, CHUNK)],
                            srcL.at[rowbase + j])
            pltpu.sync_copy(stage_d.at[pl.ds(j * CHUNK, CHUNK)],
                            dstL.at[rowbase + j])
            return carry

        lax.fori_loop(0, ng * NBUF, cpy, 0)

        cnt_v[...] = jnp.where(lane == g, ng, cnt_v[...])

    pltpu.sync_copy(cnt_v, cnt_out.at[w])


def _prep_call(src_p, dst_p):
    fn = pl.kernel(
        _prep_body,
        out_type=[
            jax.ShapeDtypeStruct((NC * NP,), jnp.float32),
            jax.ShapeDtypeStruct((NW * NGRP * T, CHUNK), jnp.int32),
            jax.ShapeDtypeStruct((NW * NGRP * T, CHUNK), jnp.int32),
            jax.ShapeDtypeStruct((NW, 16), jnp.int32),
        ],
        mesh=_mesh(),
        compiler_params=_SC_PARAMS,
        scratch_types=[
            pltpu.VMEM((T, CHUNK), jnp.int32),
            pltpu.VMEM((T, CHUNK), jnp.int32),
            pltpu.VMEM((STG,), jnp.int32),
            pltpu.VMEM((STG,), jnp.int32),
            pltpu.VMEM((16,), jnp.int32),
            pltpu.VMEM((CHUNK,), jnp.float32),
            pltpu.VMEM((NP // NS,), jnp.float32),
            pltpu.VMEM_SHARED((NP,), jnp.float32),
        ],
    )
    return fn(src_p, dst_p)


def _agg_body(h_hbm, srcL, dstL, cnt_hbm, out_hbm, src_v, dst_v, cnt_v, rows,
              acc_sh, sem0, sem1, sem2, sem3):
    sems = [sem0, sem1, sem2, sem3]
    c = lax.axis_index("c")
    s = lax.axis_index("s")
    w = c * NS + s
    row0 = s * G_ROWS
    lane = lax.iota(jnp.int32, 16)

    pltpu.sync_copy(cnt_hbm.at[w], cnt_v)

    for g in range(NGRP):
        rowbase = (w * NGRP + g) * T
        pltpu.sync_copy(srcL.at[pl.ds(rowbase, T)], src_v)
        pltpu.sync_copy(dstL.at[pl.ds(rowbase, T)], dst_v)

        def zfill(i, carry):
            r = i // (D // 16)
            k = i % (D // 16)
            rows[0, r, pl.ds(k * 16, 16)] = jnp.zeros((16,), jnp.float32)
            return carry

        lax.fori_loop(0, CHUNK * (D // 16), zfill, 0)
        for off in range(0, G_ROWS, CHUNK):
            ln = min(CHUNK, G_ROWS - off)
            pltpu.sync_copy(rows.at[0, pl.ds(0, ln)],
                            acc_sh.at[pl.ds(row0 + off, ln)])

        plsc.subcore_barrier()

        ng = lax.reduce_max(jnp.where(lane == g, cnt_v[...], 0), (0,))
        nch = ng * NBUF

        for b in range(NBUF):
            @pl.when(b < nch)
            def _():
                pltpu.async_copy(h_hbm.at[src_v.at[b]], rows.at[b], sems[b])

        def ring(gg, carry):
            for b in range(NBUF):
                j = gg * NBUF + b

                @pl.when(j < nch)
                def _():
                    pltpu.make_async_copy(h_hbm.at[src_v.at[j]], rows.at[b],
                                          sems[b]).wait()
                    pltpu.sync_copy(rows.at[b], acc_sh.at[dst_v.at[j]],
                                    add=True)

                jn = j + NBUF

                @pl.when(jn < nch)
                def _():
                    pltpu.async_copy(h_hbm.at[src_v.at[jn]], rows.at[b],
                                     sems[b])

            return carry

        lax.fori_loop(0, T // NBUF, ring, 0)

        plsc.subcore_barrier()
        for off in range(0, G_ROWS, CHUNK):
            ln = min(CHUNK, G_ROWS - off)
            r = row0 + off
            pltpu.sync_copy(acc_sh.at[pl.ds(r, ln)],
                            out_hbm.at[c, g, pl.ds(r, ln)])


def _agg_call(h, srcL, dstL, cnt):
    fn = pl.kernel(
        _agg_body,
        out_type=jax.ShapeDtypeStruct((NC, NGRP, G_ACC, D), jnp.float32),
        mesh=_mesh(),
        compiler_params=_SC_PARAMS,
        scratch_types=[
            pltpu.VMEM((T, CHUNK), jnp.int32),
            pltpu.VMEM((T, CHUNK), jnp.int32),
            pltpu.VMEM((16,), jnp.int32),
            pltpu.VMEM((NBUF, CHUNK, D), jnp.float32),
            pltpu.VMEM_SHARED((G_ACC, D), jnp.float32),
            pltpu.SemaphoreType.DMA,
            pltpu.SemaphoreType.DMA,
            pltpu.SemaphoreType.DMA,
            pltpu.SemaphoreType.DMA,
        ],
    )
    return fn(h, srcL, dstL, cnt)


def _mm_body(cur_ref, w_ref, d_ref, hp_ref):
    xw = jnp.dot(cur_ref[...], w_ref[...], preferred_element_type=jnp.float32)
    hp_ref[...] = xw * d_ref[...]


def _mm_call(cur, w, d_bcast):
    return pl.pallas_call(
        _mm_body,
        grid=(NP // RB,),
        in_specs=[
            pl.BlockSpec((RB, D), lambda i: (i, 0)),
            pl.BlockSpec((D, D), lambda i: (0, 0)),
            pl.BlockSpec((RB, D), lambda i: (i, 0)),
        ],
        out_specs=pl.BlockSpec((RB, D), lambda i: (i, 0)),
        out_shape=jax.ShapeDtypeStruct((NP, D), jnp.float32),
    )(cur, w, d_bcast)


def _comb_body(agg_ref, hp_ref, d_ref, b_ref, f_ref, out_ref):
    t = (d_ref[...] * (agg_ref[0, 0] + agg_ref[1, 0] + hp_ref[...])
         + b_ref[...])
    out_ref[...] = jnp.where(f_ref[...] > 0.0, jnp.maximum(t, 0.0), t)


def _comb_call(agg, hp, d_bcast, b, flag):
    bpg = GRP // RB
    return pl.pallas_call(
        _comb_body,
        grid=(NP // RB,),
        in_specs=[
            pl.BlockSpec((NC, 1, RB, D),
                         lambda i: (0, i // bpg, i % bpg, 0)),
            pl.BlockSpec((RB, D), lambda i: (i, 0)),
            pl.BlockSpec((RB, D), lambda i: (i, 0)),
            pl.BlockSpec((1, D), lambda i: (0, 0)),
            pl.BlockSpec((1, D), lambda i: (0, 0)),
        ],
        out_specs=pl.BlockSpec((RB, D), lambda i: (i, 0)),
        out_shape=jax.ShapeDtypeStruct((NP, D), jnp.float32),
    )(agg, hp, d_bcast, b, flag)


def kernel(x, edge_index, W1, b1, W2, b2):
    src = edge_index[0]
    dst = edge_index[1]
    e = src.shape[0]
    pad = E_PAD - e
    src_p = jnp.concatenate(
        [src, jnp.zeros((pad,), src.dtype)]).reshape(E_PAD // CHUNK, CHUNK)
    dst_p = jnp.concatenate(
        [dst, jnp.full((pad,), N, dst.dtype)]).reshape(E_PAD // CHUNK, CHUNK)

    deg_parts, srcL, dstL, cnt = _prep_call(src_p, dst_p)
    deg = deg_parts[:NP] + deg_parts[NP:] + 1.0
    d_bcast = jnp.broadcast_to(lax.rsqrt(deg)[:, None], (NP, D))

    x_pad = jnp.concatenate(
        [x, jnp.zeros((NP - N, D), jnp.float32)], axis=0)

    def layer(it, cur):
        w = jnp.where(it == 0, W1, W2)
        b = jnp.where(it == 0, b1, b2).reshape(1, D)
        flag = jnp.where(it == 0, 1.0, 0.0) * jnp.ones((1, D), jnp.float32)
        hp = _mm_call(cur, w, d_bcast)
        agg = _agg_call(hp, srcL, dstL, cnt)
        return _comb_call(agg, hp, d_bcast, b, flag)

    return lax.fori_loop(0, 2, layer, x_pad)[:N]


# speedup vs baseline: 8.2206x; 4.1006x over previous
import jax
import jax.numpy as jnp
from jax import lax
from jax.experimental import pallas as pl
from jax.experimental.pallas import tpu as pltpu
from jax.experimental.pallas import tpu_sc as plsc

N = 10000
NP = 10240
D = 128
NC = 2
NS = 16
NW = NC * NS
CHUNK = 128
T = 80
E_PAD = NW * T * CHUNK
NGRP = 2
GRP = NP // NGRP
G_ACC = GRP + CHUNK
G_ROWS = G_ACC // NS
NBUF = 4
RB = 640

_SC_PARAMS = pltpu.CompilerParams(use_tc_tiling_on_sc=False)


def _mesh():
    return plsc.VectorSubcoreMesh(core_axis_name="c", subcore_axis_name="s")


def _deg_body(dst_hbm, deg_out, dst_v, ones_v, zero_v, acc_sh):
    c = lax.axis_index("c")
    s = lax.axis_index("s")
    row0 = s * (NP // NS)

    def zfill(i, carry):
        zero_v[pl.ds(i * 16, 16)] = jnp.zeros((16,), jnp.float32)
        return carry

    lax.fori_loop(0, (NP // NS) // 16, zfill, 0)
    pltpu.sync_copy(zero_v, acc_sh.at[pl.ds(row0, NP // NS)])

    def ofill(i, carry):
        ones_v[pl.ds(i * 16, 16)] = jnp.ones((16,), jnp.float32)
        return carry

    lax.fori_loop(0, CHUNK // 16, ofill, 0)

    tbase = c * (NS * T) + s * T
    pltpu.sync_copy(dst_hbm.at[pl.ds(tbase, T)], dst_v)

    plsc.subcore_barrier()

    def body(j, carry):
        pltpu.sync_copy(ones_v, acc_sh.at[dst_v.at[j]], add=True)
        return carry

    lax.fori_loop(0, T, body, 0)

    plsc.subcore_barrier()
    pltpu.sync_copy(acc_sh.at[pl.ds(row0, NP // NS)],
                    deg_out.at[pl.ds(c * NP + row0, NP // NS)])


def _deg_call(dst_p):
    fn = pl.kernel(
        _deg_body,
        out_type=jax.ShapeDtypeStruct((NC * NP,), jnp.float32),
        mesh=_mesh(),
        compiler_params=_SC_PARAMS,
        scratch_types=[
            pltpu.VMEM((T, CHUNK), jnp.int32),
            pltpu.VMEM((CHUNK,), jnp.float32),
            pltpu.VMEM((NP // NS,), jnp.float32),
            pltpu.VMEM_SHARED((NP,), jnp.float32),
        ],
    )
    return fn(dst_p)


def _agg_body(h_hbm, src_hbm, dst_hbm, out_hbm, src_v, dst_v, dst_a, rows,
              acc_sh, *sems):
    c = lax.axis_index("c")
    s = lax.axis_index("s")
    row0 = s * G_ROWS

    tbase = c * (NS * T) + s * T
    pltpu.sync_copy(src_hbm.at[pl.ds(tbase, T)], src_v)
    pltpu.sync_copy(dst_hbm.at[pl.ds(tbase, T)], dst_v)

    for g in range(NGRP):
        gbase = g * GRP

        def adjust(i, carry):
            r = i // (CHUNK // 16)
            k = i % (CHUNK // 16)
            v = dst_v[r, pl.ds(k * 16, 16)] - gbase
            ok = (v >= 0) & (v < GRP)
            dump = GRP + (i % CHUNK)
            dst_a[r, pl.ds(k * 16, 16)] = jnp.where(ok, v, dump)
            return carry

        lax.fori_loop(0, T * (CHUNK // 16), adjust, 0)

        def zfill(i, carry):
            r = i // (D // 32)
            k = i % (D // 32)
            rows[0, r, pl.ds(k * 32, 32)] = jnp.zeros((32,), jnp.bfloat16)
            return carry

        lax.fori_loop(0, CHUNK * (D // 32), zfill, 0)
        for off in range(0, G_ROWS, CHUNK):
            ln = min(CHUNK, G_ROWS - off)
            pltpu.sync_copy(rows.at[0, pl.ds(0, ln)],
                            acc_sh.at[pl.ds(row0 + off, ln)])

        plsc.subcore_barrier()

        for b in range(NBUF):
            pltpu.async_copy(h_hbm.at[src_v.at[b]], rows.at[b], sems[b])

        def group(gg, carry):
            for b in range(NBUF):
                j = gg * NBUF + b
                pltpu.make_async_copy(h_hbm.at[src_v.at[j]], rows.at[b],
                                      sems[b]).wait()
                pltpu.sync_copy(rows.at[b], acc_sh.at[dst_a.at[j]], add=True)
                jn = j + NBUF

                @pl.when(jn < T)
                def _():
                    pltpu.async_copy(h_hbm.at[src_v.at[jn]], rows.at[b],
                                     sems[b])

            return carry

        lax.fori_loop(0, T // NBUF, group, 0)

        plsc.subcore_barrier()
        for off in range(0, G_ROWS, CHUNK):
            ln = min(CHUNK, G_ROWS - off)
            r = row0 + off
            pltpu.sync_copy(acc_sh.at[pl.ds(r, ln)],
                            out_hbm.at[c, g, pl.ds(r, ln)])


def _agg_call(h, src_p, dst_p):
    fn = pl.kernel(
        _agg_body,
        out_type=jax.ShapeDtypeStruct((NC, NGRP, G_ACC, D), jnp.bfloat16),
        mesh=_mesh(),
        compiler_params=_SC_PARAMS,
        scratch_types=[
            pltpu.VMEM((T, CHUNK), jnp.int32),
            pltpu.VMEM((T, CHUNK), jnp.int32),
            pltpu.VMEM((T, CHUNK), jnp.int32),
            pltpu.VMEM((NBUF, CHUNK, D), jnp.bfloat16),
            pltpu.VMEM_SHARED((G_ACC, D), jnp.bfloat16),
        ] + [pltpu.SemaphoreType.DMA] * NBUF,
    )
    return fn(h, src_p, dst_p)


def _mm_body(cur_ref, w_ref, d_ref, hp_ref, hpf_ref):
    xw = jnp.dot(cur_ref[...], w_ref[...], preferred_element_type=jnp.float32)
    hp = xw * d_ref[...]
    hp_ref[...] = hp.astype(jnp.bfloat16)
    hpf_ref[...] = hp


def _mm_call(cur, w, d_bcast):
    return pl.pallas_call(
        _mm_body,
        grid=(NP // RB,),
        in_specs=[
            pl.BlockSpec((RB, D), lambda i: (i, 0)),
            pl.BlockSpec((D, D), lambda i: (0, 0)),
            pl.BlockSpec((RB, D), lambda i: (i, 0)),
        ],
        out_specs=[
            pl.BlockSpec((RB, D), lambda i: (i, 0)),
            pl.BlockSpec((RB, D), lambda i: (i, 0)),
        ],
        out_shape=[
            jax.ShapeDtypeStruct((NP, D), jnp.bfloat16),
            jax.ShapeDtypeStruct((NP, D), jnp.float32),
        ],
    )(cur, w, d_bcast)


def _comb_body(agg_ref, hpf_ref, d_ref, b_ref, f_ref, out_ref):
    a = (agg_ref[0, 0].astype(jnp.float32)
         + agg_ref[1, 0].astype(jnp.float32))
    t = d_ref[...] * (a + hpf_ref[...]) + b_ref[...]
    out_ref[...] = jnp.where(f_ref[...] > 0.0, jnp.maximum(t, 0.0), t)


def _comb_call(agg, hpf, d_bcast, b, flag):
    bpg = GRP // RB
    return pl.pallas_call(
        _comb_body,
        grid=(NP // RB,),
        in_specs=[
            pl.BlockSpec((NC, 1, RB, D),
                         lambda i: (0, i // bpg, i % bpg, 0)),
            pl.BlockSpec((RB, D), lambda i: (i, 0)),
            pl.BlockSpec((RB, D), lambda i: (i, 0)),
            pl.BlockSpec((1, D), lambda i: (0, 0)),
            pl.BlockSpec((1, D), lambda i: (0, 0)),
        ],
        out_specs=pl.BlockSpec((RB, D), lambda i: (i, 0)),
        out_shape=jax.ShapeDtypeStruct((NP, D), jnp.float32),
    )(agg, hpf, d_bcast, b, flag)


def kernel(x, edge_index, W1, b1, W2, b2):
    src = edge_index[0]
    dst = edge_index[1]
    e = src.shape[0]
    pad = E_PAD - e
    src_p = jnp.concatenate(
        [src, jnp.zeros((pad,), src.dtype)]).reshape(E_PAD // CHUNK, CHUNK)
    dst_p = jnp.concatenate(
        [dst, jnp.full((pad,), N, dst.dtype)]).reshape(E_PAD // CHUNK, CHUNK)

    deg_parts = _deg_call(dst_p)
    deg = deg_parts[:NP] + deg_parts[NP:] + 1.0
    d_bcast = jnp.broadcast_to(lax.rsqrt(deg)[:, None], (NP, D))

    x_pad = jnp.concatenate(
        [x, jnp.zeros((NP - N, D), jnp.float32)], axis=0)

    def layer(it, cur):
        w = jnp.where(it == 0, W1, W2)
        b = jnp.where(it == 0, b1, b2).reshape(1, D)
        flag = jnp.where(it == 0, 1.0, 0.0) * jnp.ones((1, D), jnp.float32)
        hp, hpf = _mm_call(cur, w, d_bcast)
        agg = _agg_call(hp, src_p, dst_p)
        return _comb_call(agg, hpf, d_bcast, b, flag)

    return lax.fori_loop(0, 2, layer, x_pad)[:N]
